# Initial kernel scaffold; baseline (speedup 1.0000x reference)
import jax
import jax.numpy as jnp
from jax.experimental import pallas as pl


def kernel(x, a_k, b_k):
    raise NotImplementedError("write your pallas kernel here")


# speedup vs baseline: 1.6266x; 1.6266x over previous
from functools import partial

import jax
import jax.numpy as jnp
from jax.experimental import pallas as pl
from jax.experimental.pallas import tpu as pltpu

_K = 99074
_ITERS = 31


def _haar_bits_kernel(a_ref, b_ref, d_ref, e_ref, out_ref):
    a = a_ref[0]
    b = b_ref[0]
    d = d_ref[0]
    e = e_ref[0]
    b2 = jnp.concatenate([b[:, :, -1:], b[:, :, :-1]], axis=2)
    d2 = jnp.concatenate([d[:, -1:, :], d[:, :-1, :]], axis=1)
    e1 = jnp.concatenate([e[:, -1:, :], e[:, :-1, :]], axis=1)
    e2 = jnp.concatenate([e1[:, :, -1:], e1[:, :, :-1]], axis=2)
    c = 0.5 * (e2 - d2 - b2 + a)
    bits = jax.lax.bitcast_convert_type(jnp.abs(c), jnp.int32)
    out_ref[0] = bits.reshape(768, 256)


def _select_kernel(bits_ref, sp_ref, out_ref):
    bits = bits_ref[...]
    row = jax.lax.broadcasted_iota(jnp.int32, (1, 768, 256), 1)
    col = jax.lax.broadcasted_iota(jnp.int32, (1, 768, 256), 2)
    wi = jnp.where((row % 256) == 0, 2, 1)
    wj = jnp.where(col == 0, 2, 1)
    w = wi * wj

    def body(_, carry):
        lo, hi = carry
        mid = lo + (hi - lo + 1) // 2
        cnt = jnp.sum(jnp.where(bits >= mid, w, 0), axis=(1, 2), keepdims=True)
        ge = cnt >= _K
        return jnp.where(ge, mid, lo), jnp.where(ge, hi, mid - 1)

    lo0 = jnp.zeros((16, 1, 1), jnp.int32)
    hi0 = jnp.full((16, 1, 1), 0x7FFFFFFE, jnp.int32)
    lo, _ = jax.lax.fori_loop(0, _ITERS, body, (lo0, hi0))
    med = jax.lax.bitcast_convert_type(lo, jnp.float32)
    std = med / 0.6745
    sp_a = sp_ref[0, 0]
    sp_b = sp_ref[0, 1]
    beta = 1.0 / (sp_a * std + sp_b) ** 2
    out_ref[...] = jnp.broadcast_to(beta.reshape(16, 1), (16, 128))


def kernel(x, a_k, b_k):
    x = x.astype(jnp.float32)
    a = x[:, :, 0::2, 0::2]
    b = x[:, :, 0::2, 1::2]
    d = x[:, :, 1::2, 0::2]
    e = x[:, :, 1::2, 1::2]

    blk = pl.BlockSpec((1, 3, 256, 256), lambda i: (i, 0, 0, 0))
    bits = pl.pallas_call(
        _haar_bits_kernel,
        grid=(16,),
        in_specs=[blk, blk, blk, blk],
        out_specs=pl.BlockSpec((1, 768, 256), lambda i: (i, 0, 0)),
        out_shape=jax.ShapeDtypeStruct((16, 768, 256), jnp.int32),
    )(a, b, d, e)

    sp = jax.nn.softplus(jnp.stack([a_k, b_k])).reshape(1, 2)
    out = pl.pallas_call(
        _select_kernel,
        in_specs=[
            pl.BlockSpec(memory_space=pltpu.VMEM),
            pl.BlockSpec(memory_space=pltpu.SMEM),
        ],
        out_shape=jax.ShapeDtypeStruct((16, 128), jnp.float32),
    )(bits, sp)
    return out[:, 0]


# speedup vs baseline: 19.7219x; 12.1247x over previous
import jax
import jax.numpy as jnp
from jax.experimental import pallas as pl
from jax.experimental.pallas import tpu as pltpu

_K = 99074
_ITERS = 31
_CHUNK = 96


def _haar_bits_kernel(x_ref, out_ref):
    v = x_ref[0, 0]
    rp = jax.lax.broadcasted_iota(jnp.int32, (512, 512), 0)
    cp = jax.lax.broadcasted_iota(jnp.int32, (512, 512), 1)
    sign = jnp.where(((rp ^ cp) & 1) == 0, 0.5, -0.5)
    z = v * sign
    t = z + jnp.concatenate([z[:, -1:], z[:, :-1]], axis=1)
    u = t + jnp.concatenate([t[-1:, :], t[:-1, :]], axis=0)
    ue = u.reshape(256, 2, 512)[:, 0, :]
    out_ref[0, 0] = jax.lax.bitcast_convert_type(jnp.abs(ue), jnp.int32)


def _select_kernel(bits_ref, sp_ref, out_ref):
    def count(mid):
        def chunk_body(c, acc):
            blk = bits_ref[:, pl.ds(c * _CHUNK, _CHUNK), :]
            row = jax.lax.broadcasted_iota(
                jnp.int32, (1, _CHUNK, 512), 1) + c * _CHUNK
            col = jax.lax.broadcasted_iota(jnp.int32, (1, _CHUNK, 512), 2)
            w = jnp.where(row % 256 == 0, 2, 1) * jnp.where(col == 0, 2, 1)
            w = jnp.where(col % 2 == 0, w, 0)
            return acc + jnp.sum(jnp.where(blk >= mid, w, 0), axis=(1, 2),
                                 keepdims=True)
        acc0 = jnp.zeros((16, 1, 1), jnp.int32)
        return jax.lax.fori_loop(0, 768 // _CHUNK, chunk_body, acc0)

    def body(_, carry):
        lo, hi = carry
        mid = lo + (hi - lo + 1) // 2
        ge = count(mid) >= _K
        return jnp.where(ge, mid, lo), jnp.where(ge, hi, mid - 1)

    lo0 = jnp.zeros((16, 1, 1), jnp.int32)
    hi0 = jnp.full((16, 1, 1), 0x7FFFFFFE, jnp.int32)
    lo, _ = jax.lax.fori_loop(0, _ITERS, body, (lo0, hi0))
    med = jax.lax.bitcast_convert_type(lo, jnp.float32)
    std = med / 0.6745
    sp_a = sp_ref[0, 0]
    sp_b = sp_ref[0, 1]
    beta = 1.0 / (sp_a * std + sp_b) ** 2
    out_ref[...] = jnp.broadcast_to(beta.reshape(16, 1), (16, 128))


def kernel(x, a_k, b_k):
    x = x.astype(jnp.float32)

    bits = pl.pallas_call(
        _haar_bits_kernel,
        grid=(16, 3),
        in_specs=[pl.BlockSpec((1, 1, 512, 512), lambda i, j: (i, j, 0, 0))],
        out_specs=pl.BlockSpec((1, 1, 256, 512), lambda i, j: (i, j, 0, 0)),
        out_shape=jax.ShapeDtypeStruct((16, 3, 256, 512), jnp.int32),
    )(x)

    sp = jax.nn.softplus(jnp.stack([a_k, b_k])).reshape(1, 2)
    out = pl.pallas_call(
        _select_kernel,
        in_specs=[
            pl.BlockSpec(memory_space=pltpu.VMEM),
            pl.BlockSpec(memory_space=pltpu.SMEM),
        ],
        out_shape=jax.ShapeDtypeStruct((16, 128), jnp.float32),
    )(bits.reshape(16, 768, 512), sp)
    return out[:, 0]
